# Initial kernel scaffold; baseline (speedup 1.0000x reference)
import jax
import jax.numpy as jnp
from jax.experimental import pallas as pl


def kernel(loc_preds, cls_preds, anchors, gt_boxes, gt_labels):
    raise NotImplementedError("write your pallas kernel here")


# speedup vs baseline: 12.8925x; 12.8925x over previous
import jax
import jax.numpy as jnp
from jax.experimental import pallas as pl

_VAR0, _VAR1 = 0.1, 0.2
_IOU_THR = 0.5
_NEG_POS = 3
_EPS = 1e-7


def _match_body(anc_ref, gt_ref, lab_ref, locp_ref, clst_ref, locl_ref):
    M = gt_ref.shape[1]
    N = anc_ref.shape[1]
    acx = anc_ref[0:1, :]
    acy = anc_ref[1:2, :]
    aw = anc_ref[2:3, :]
    ah = anc_ref[3:4, :]
    ax1 = acx - aw * 0.5
    ay1 = acy - ah * 0.5
    ax2 = acx + aw * 0.5
    ay2 = acy + ah * 0.5
    area_a = (ax2 - ax1) * (ay2 - ay1)

    gt = gt_ref[0]
    gx1 = gt[:, 0:1]
    gy1 = gt[:, 1:2]
    gx2 = gt[:, 2:3]
    gy2 = gt[:, 3:4]
    area_g = (gx2 - gx1) * (gy2 - gy1)

    ix1 = jnp.maximum(ax1, gx1)
    iy1 = jnp.maximum(ay1, gy1)
    ix2 = jnp.minimum(ax2, gx2)
    iy2 = jnp.minimum(ay2, gy2)
    iw = jnp.clip(ix2 - ix1, 0.0, None)
    ih = jnp.clip(iy2 - iy1, 0.0, None)
    inter = iw * ih
    union = area_a + area_g - inter
    iou = inter / jnp.clip(union, 1e-6, None)

    jio = jax.lax.broadcasted_iota(jnp.int32, (M, N), 0)
    lio = jax.lax.broadcasted_iota(jnp.int32, (M, N), 1)

    best_iou = jnp.max(iou, axis=0, keepdims=True)
    best_j = jnp.min(jnp.where(iou == best_iou, jio, M), axis=0, keepdims=True)
    colmax = jnp.max(iou, axis=1, keepdims=True)
    best_i = jnp.min(jnp.where(iou == colmax, lio, N), axis=1, keepdims=True)
    forced = jnp.max(jnp.where(best_i == lio, jio, -1), axis=0, keepdims=True)
    bj = jnp.where(forced >= 0, forced, best_j)
    biou = jnp.where(forced >= 0, 1.0, best_iou)
    pos = biou >= _IOU_THR

    onehot = bj == jio
    lab = lab_ref[0]
    cls_t = jnp.where(pos, jnp.sum(jnp.where(onehot, lab, 0), axis=0, keepdims=True), 0)
    clst_ref[0] = cls_t

    mgx1 = jnp.sum(jnp.where(onehot, gx1, 0.0), axis=0, keepdims=True)
    mgy1 = jnp.sum(jnp.where(onehot, gy1, 0.0), axis=0, keepdims=True)
    mgx2 = jnp.sum(jnp.where(onehot, gx2, 0.0), axis=0, keepdims=True)
    mgy2 = jnp.sum(jnp.where(onehot, gy2, 0.0), axis=0, keepdims=True)
    mcx = (mgx1 + mgx2) * 0.5
    mcy = (mgy1 + mgy2) * 0.5
    mw = mgx2 - mgx1
    mh = mgy2 - mgy1

    dcx = (mcx - acx) / (_VAR0 * aw)
    dcy = (mcy - acy) / (_VAR0 * ah)
    dw = jnp.log(jnp.clip(mw / jnp.clip(aw, _EPS, None), _EPS, None)) / _VAR1
    dh = jnp.log(jnp.clip(mh / jnp.clip(ah, _EPS, None), _EPS, None)) / _VAR1

    lp = locp_ref[0]
    total = jnp.zeros((1, 1), jnp.float32)
    for c, tgt in enumerate((dcx, dcy, dw, dh)):
        d = lp[c : c + 1, :] - jnp.where(pos, tgt, 0.0)
        ad = jnp.abs(d)
        sl1 = jnp.where(ad < 1.0, 0.5 * ad * ad, ad - 0.5)
        total = total + jnp.sum(jnp.where(pos, sl1, 0.0), axis=1, keepdims=True)
    locl_ref[0] = total


def _ce_body(x_ref, t_ref, ce_ref):
    x = x_ref[0]
    t = t_ref[0]
    m = jnp.max(x, axis=1, keepdims=True)
    lse = m + jnp.log(jnp.sum(jnp.exp(x - m), axis=1, keepdims=True))
    cio = jax.lax.broadcasted_iota(jnp.int32, x.shape, 1)
    picked = jnp.sum(jnp.where(cio == t, x, 0.0), axis=1, keepdims=True)
    ce_ref[0] = lse - picked


def _combine_body(ce_ref, t_ref, locl_ref, out_ref):
    ce = ce_ref[...]
    tgt = t_ref[...]
    B, N = ce.shape
    pos = tgt > 0
    npos_b = jnp.sum(pos.astype(jnp.int32), axis=1, keepdims=True)
    pos_ce = jnp.sum(jnp.where(pos, ce, 0.0), axis=1, keepdims=True)
    ceneg = jnp.where(pos, 0.0, ce)
    bits = jax.lax.bitcast_convert_type(ceneg, jnp.int32)
    k = jnp.minimum(_NEG_POS * npos_b, N - 1)

    def bs1(_, lohi):
        lo, hi = lohi
        mid = lo + (hi - lo + 1) // 2
        cnt = jnp.sum((bits >= mid).astype(jnp.int32), axis=1, keepdims=True)
        ok = cnt >= k
        return jnp.where(ok, mid, lo), jnp.where(ok, hi, mid)

    lo0 = jnp.zeros((B, 1), jnp.int32)
    hi0 = jnp.full((B, 1), jnp.int32(0x7F800001))
    tbits, _ = jax.lax.fori_loop(0, 31, bs1, (lo0, hi0))

    cnt_gt = jnp.sum((bits > tbits).astype(jnp.int32), axis=1, keepdims=True)
    sum_gt = jnp.sum(jnp.where(bits > tbits, ce, 0.0), axis=1, keepdims=True)
    r = k - cnt_gt

    tie = bits == tbits
    lane = jax.lax.broadcasted_iota(jnp.int32, (B, N), 1)

    def bs2(_, lohi):
        lo, hi = lohi
        mid = (lo + hi) // 2
        g = jnp.sum((tie & (lane < mid)).astype(jnp.int32), axis=1, keepdims=True)
        ok = g >= r
        return jnp.where(ok, lo, mid + 1), jnp.where(ok, mid, hi)

    lo0b = jnp.zeros((B, 1), jnp.int32)
    hi0b = jnp.full((B, 1), N)
    mstar, _ = jax.lax.fori_loop(0, 15, bs2, (lo0b, hi0b))

    sum_tie = jnp.sum(
        jnp.where(tie & (lane < mstar), ce, 0.0), axis=1, keepdims=True
    )
    cls_loss = jnp.sum(
        pos_ce + sum_gt + jnp.where(r > 0, sum_tie, 0.0), axis=0, keepdims=True
    )
    loc_loss = jnp.sum(locl_ref[...], axis=0, keepdims=True)
    npos = jnp.maximum(jnp.sum(npos_b, axis=0, keepdims=True), 1).astype(jnp.float32)
    out_ref[...] = (loc_loss + cls_loss) / npos


def kernel(loc_preds, cls_preds, anchors, gt_boxes, gt_labels):
    B, N, C = cls_preds.shape
    M = gt_boxes.shape[1]
    anc_t = jnp.transpose(anchors, (1, 0))
    locp_t = jnp.transpose(loc_preds, (0, 2, 1))
    lab3 = gt_labels.astype(jnp.int32)[..., None]

    cls_t, loc_l = pl.pallas_call(
        _match_body,
        grid=(B,),
        in_specs=[
            pl.BlockSpec((4, N), lambda b: (0, 0)),
            pl.BlockSpec((1, M, 4), lambda b: (b, 0, 0)),
            pl.BlockSpec((1, M, 1), lambda b: (b, 0, 0)),
            pl.BlockSpec((1, 4, N), lambda b: (b, 0, 0)),
        ],
        out_specs=[
            pl.BlockSpec((1, 1, N), lambda b: (b, 0, 0)),
            pl.BlockSpec((1, 1, 1), lambda b: (b, 0, 0)),
        ],
        out_shape=[
            jax.ShapeDtypeStruct((B, 1, N), jnp.int32),
            jax.ShapeDtypeStruct((B, 1, 1), jnp.float32),
        ],
    )(anc_t, gt_boxes, lab3, locp_t)

    Nb = 2000
    cls_t_col = cls_t.reshape(B, N, 1)
    ce = pl.pallas_call(
        _ce_body,
        grid=(B, N // Nb),
        in_specs=[
            pl.BlockSpec((1, Nb, C), lambda b, n: (b, n, 0)),
            pl.BlockSpec((1, Nb, 1), lambda b, n: (b, n, 0)),
        ],
        out_specs=pl.BlockSpec((1, Nb, 1), lambda b, n: (b, n, 0)),
        out_shape=jax.ShapeDtypeStruct((B, N, 1), jnp.float32),
    )(cls_preds, cls_t_col)

    out = pl.pallas_call(
        _combine_body,
        in_specs=[
            pl.BlockSpec((B, N), lambda: (0, 0)),
            pl.BlockSpec((B, N), lambda: (0, 0)),
            pl.BlockSpec((B, 1), lambda: (0, 0)),
        ],
        out_specs=pl.BlockSpec((1, 1), lambda: (0, 0)),
        out_shape=jax.ShapeDtypeStruct((1, 1), jnp.float32),
    )(ce.reshape(B, N), cls_t.reshape(B, N), loc_l.reshape(B, 1))
    return out[0, 0]
